# baseline (speedup 1.0000x reference)
import functools

import jax
import jax.numpy as jnp
from jax import lax
from jax.experimental import pallas as pl
from jax.experimental.pallas import tpu as pltpu
from jax.experimental.pallas import tpu_sc as plsc

N = 10000
E = 320000
F = 128
HID = 128
C = 40
CP = 64

NC = 2
NS = 16
NW = NC * NS
CHUNK = 64
NCH = 160
EPT = NCH * CHUNK
EPAD = EPT * NW
NP = 10240
RPT = NP // NS
NB = 8
SLAB = 32
TOTCH = EPT * NW // CHUNK
CNT0 = 288
CNT1 = 32
NPH0 = max(CNT0, CNT1) // SLAB

BR = 2000


def _deg_body(dst_hbm, out_hbm, idx_v, deg_v):
    c = lax.axis_index("c")
    s = lax.axis_index("s")
    w = c * NS + s
    pltpu.sync_copy(dst_hbm.at[w], idx_v)
    zeros16 = jnp.zeros((16,), jnp.float32)
    ones16 = jnp.ones((16,), jnp.float32)

    def zero_step(i, carry):
        deg_v[pl.ds(i * 16, 16)] = zeros16
        return carry

    lax.fori_loop(0, NP // 16, zero_step, 0)

    def count_step(j, carry):
        for k in range(CHUNK // 16):
            idx = idx_v[j, pl.ds(k * 16, 16)]
            plsc.addupdate_scatter(deg_v, [idx], ones16)
        return carry

    lax.fori_loop(0, NCH, count_step, 0)
    pltpu.sync_copy(deg_v, out_hbm.at[w])


_deg_call = functools.partial(
    pl.kernel,
    out_type=jax.ShapeDtypeStruct((NW, NP), jnp.float32),
    mesh=plsc.VectorSubcoreMesh(core_axis_name="c", subcore_axis_name="s"),
    scratch_types=[
        pltpu.VMEM((NCH, CHUNK), jnp.int32),
        pltpu.VMEM((NP,), jnp.float32),
    ],
    compiler_params=pltpu.CompilerParams(needs_layout_passes=False),
)(_deg_body)


def _make_agg(D, chunk, cnt0, cnt1, slab, nb):
    nph = max(cnt0, cnt1) // slab

    def body(tbl_hbm, src_hbm, dst_hbm, out_hbm, src_v, dst_v, rows_v, acc,
             sems):
        c = lax.axis_index("c")
        s = lax.axis_index("s")

        zeros32 = jnp.zeros((32,), jnp.bfloat16)

        def zero_step(i, carry):
            for k in range(D // 32):
                rows_v[0, i, pl.ds(k * 32, 32)] = zeros32
            return carry

        with jax.named_scope("zero_acc"):
            lax.fori_loop(0, chunk, zero_step, 0)
            for k in range(RPT // chunk):
                pltpu.sync_copy(rows_v.at[0],
                                acc.at[pl.ds(s * RPT + k * chunk, chunk)])
            plsc.subcore_barrier()

        cnt = jnp.where(c == 0, cnt0, cnt1)
        base = c * (NS * cnt0) + s * cnt

        def run_phase(p):
            pltpu.sync_copy(src_hbm.at[pl.ds(base + p * slab, slab)], src_v)
            pltpu.sync_copy(dst_hbm.at[pl.ds(base + p * slab, slab)], dst_v)
            for b in range(nb):
                pltpu.async_copy(tbl_hbm.at[src_v.at[b]], rows_v.at[b],
                                 sems.at[b])

            def step(i, carry):
                j = i * nb
                for b in range(nb):
                    jj = j + b
                    pltpu.make_async_copy(tbl_hbm.at[src_v.at[jj]],
                                          rows_v.at[b], sems.at[b]).wait()
                    pltpu.sync_copy(rows_v.at[b], acc.at[dst_v.at[jj]],
                                    add=True)

                    @pl.when(jj + nb < slab)
                    def _():
                        pltpu.async_copy(tbl_hbm.at[src_v.at[jj + nb]],
                                         rows_v.at[b], sems.at[b])
                return carry

            lax.fori_loop(0, slab // nb, step, 0)

        with jax.named_scope("edges"):
            for p in range(nph):
                @pl.when(p * slab < cnt)
                def _():
                    run_phase(p)
            plsc.subcore_barrier()
        with jax.named_scope("dump"):
            for k in range(RPT // chunk):
                r = s * RPT + k * chunk
                pltpu.sync_copy(acc.at[pl.ds(r, chunk)],
                                out_hbm.at[c, pl.ds(r, chunk)])

    return pl.kernel(
        body,
        out_type=jax.ShapeDtypeStruct((NC, NP, D), jnp.bfloat16),
        mesh=plsc.VectorSubcoreMesh(core_axis_name="c", subcore_axis_name="s"),
        scratch_types=[
            pltpu.VMEM((slab, chunk), jnp.int32),
            pltpu.VMEM((slab, chunk), jnp.int32),
            pltpu.VMEM((nb, chunk, D), jnp.bfloat16),
            pltpu.VMEM_SHARED((NP, D), jnp.bfloat16),
            pltpu.SemaphoreType.DMA((nb,)),
        ],
        compiler_params=pltpu.CompilerParams(use_tc_tiling_on_sc=False),
    )


CH_AGG = 128
AGG_CNT0 = 160
AGG_CNT1 = 0
_agg_h = _make_agg(HID, CH_AGG, AGG_CNT0, AGG_CNT1, 16, 8)
_agg_c = _make_agg(CP, CH_AGG, AGG_CNT0, AGG_CNT1, 16, 8)


def _dsc_body(deg_ref, out_ref):
    deg = jnp.sum(deg_ref[...], axis=0, keepdims=True) + 1.0
    out_ref[...] = lax.rsqrt(deg)


def _l1_body(x_ref, dsc_ref, w1_ref, out_ref):
    xs = x_ref[...] * dsc_ref[...]
    h = jnp.dot(xs, w1_ref[...], preferred_element_type=jnp.float32)
    out_ref[...] = h.astype(jnp.bfloat16)


def _l2_body(agg_ref, sh_ref, dsc_ref, b1_ref, w2_ref, out_ref):
    dsc = dsc_ref[...]
    tot = (agg_ref[0].astype(jnp.float32) + agg_ref[1].astype(jnp.float32)
           + sh_ref[...].astype(jnp.float32))
    h1 = jnp.maximum(dsc * tot + b1_ref[...], 0.0)
    p = jnp.dot(h1 * dsc, w2_ref[...], preferred_element_type=jnp.float32)
    out_ref[...] = p.astype(jnp.bfloat16)


def _out_body(agg_ref, sp_ref, dsc_ref, b2_ref, out_ref):
    tot = (agg_ref[0].astype(jnp.float32) + agg_ref[1].astype(jnp.float32)
           + sp_ref[...].astype(jnp.float32))
    o = dsc_ref[...] * tot + b2_ref[...]
    logits = o[:, :C]
    m = jnp.max(logits, axis=1, keepdims=True)
    lse = jnp.log(jnp.sum(jnp.exp(logits - m), axis=1, keepdims=True))
    out_ref[...] = logits - m - lse


def _row_spec(d):
    return pl.BlockSpec((BR, d), lambda i: (i, 0))


def _full_spec(shape):
    return pl.BlockSpec(shape, lambda i: (0,) * len(shape))


def kernel(x, edge_index, W1, b1, W2, b2):
    src = edge_index[0].astype(jnp.int32)
    dst = edge_index[1].astype(jnp.int32)
    pad = EPAD - E
    srcp = jnp.concatenate([src, jnp.zeros((pad,), jnp.int32)])
    srcp = srcp.reshape(NW, NCH, CHUNK)
    dstp = jnp.concatenate([dst, jnp.full((pad,), N, jnp.int32)])
    dstp = dstp.reshape(NW, NCH, CHUNK)
    srcf = srcp.reshape(EPAD // CH_AGG, CH_AGG)
    dstf = dstp.reshape(EPAD // CH_AGG, CH_AGG)
    w2p = jnp.pad(W2, ((0, 0), (0, CP - C)))
    b1r = b1.reshape(1, HID)
    b2r = jnp.pad(b2, (0, CP - C)).reshape(1, CP)

    degp = _deg_call(dstp)
    dsc_row = pl.pallas_call(
        _dsc_body,
        out_shape=jax.ShapeDtypeStruct((1, NP), jnp.float32),
    )(degp)
    dsc = dsc_row.reshape(NP, 1)

    scaled_h = pl.pallas_call(
        _l1_body,
        grid=(N // BR,),
        in_specs=[_row_spec(F), _row_spec(1), _full_spec((F, HID))],
        out_specs=_row_spec(HID),
        out_shape=jax.ShapeDtypeStruct((N, HID), jnp.bfloat16),
    )(x, dsc, W1)

    agg1 = _agg_h(scaled_h, srcf, dstf)

    scaled_p = pl.pallas_call(
        _l2_body,
        grid=(N // BR,),
        in_specs=[pl.BlockSpec((NC, BR, HID), lambda i: (0, i, 0)),
                  _row_spec(HID), _row_spec(1),
                  _full_spec((1, HID)), _full_spec((HID, CP))],
        out_specs=_row_spec(CP),
        out_shape=jax.ShapeDtypeStruct((N, CP), jnp.bfloat16),
    )(agg1, scaled_h, dsc, b1r, w2p)

    agg2 = _agg_c(scaled_p, srcf, dstf)

    out = pl.pallas_call(
        _out_body,
        grid=(N // BR,),
        in_specs=[pl.BlockSpec((NC, BR, CP), lambda i: (0, i, 0)),
                  _row_spec(CP), _row_spec(1),
                  _full_spec((1, CP))],
        out_specs=_row_spec(C),
        out_shape=jax.ShapeDtypeStruct((N, C), jnp.float32),
    )(agg2, scaled_p, dsc, b2r)
    return out


# speedup vs baseline: 1.2727x; 1.2727x over previous
import functools

import jax
import jax.numpy as jnp
from jax import lax
from jax.experimental import pallas as pl
from jax.experimental.pallas import tpu as pltpu
from jax.experimental.pallas import tpu_sc as plsc

N = 10000
E = 320000
F = 128
HID = 128
C = 40
CP = 64

NC = 2
NS = 16
NW = NC * NS
CHUNK = 64
NCH = 160
EPT = NCH * CHUNK
EPAD = EPT * NW
NP = 10240
RPT = NP // NS
NB = 8
SLAB = 32
TOTCH = EPT * NW // CHUNK
CNT0 = 288
CNT1 = 32
NPH0 = max(CNT0, CNT1) // SLAB

BR = 2000


def _deg_body(dst_hbm, out_hbm, idx_v, deg_v):
    c = lax.axis_index("c")
    s = lax.axis_index("s")
    w = c * NS + s
    pltpu.sync_copy(dst_hbm.at[w], idx_v)
    zeros16 = jnp.zeros((16,), jnp.float32)
    ones16 = jnp.ones((16,), jnp.float32)

    def zero_step(i, carry):
        deg_v[pl.ds(i * 16, 16)] = zeros16
        return carry

    lax.fori_loop(0, NP // 16, zero_step, 0)

    def count_step(j, carry):
        for k in range(CHUNK // 16):
            idx = idx_v[j, pl.ds(k * 16, 16)]
            plsc.addupdate_scatter(deg_v, [idx], ones16)
        return carry

    lax.fori_loop(0, NCH, count_step, 0)
    pltpu.sync_copy(deg_v, out_hbm.at[w])


_deg_call = functools.partial(
    pl.kernel,
    out_type=jax.ShapeDtypeStruct((NW, NP), jnp.float32),
    mesh=plsc.VectorSubcoreMesh(core_axis_name="c", subcore_axis_name="s"),
    scratch_types=[
        pltpu.VMEM((NCH, CHUNK), jnp.int32),
        pltpu.VMEM((NP,), jnp.float32),
    ],
    compiler_params=pltpu.CompilerParams(needs_layout_passes=False),
)(_deg_body)


def _make_agg(D, chunk, cnt0, cnt1, slab, nb):
    nph = max(cnt0, cnt1) // slab

    def body(tbl_hbm, src_hbm, dst_hbm, out_hbm, src_v, dst_v, rows_v, acc,
             sems):
        c = lax.axis_index("c")
        s = lax.axis_index("s")

        zeros32 = jnp.zeros((32,), jnp.bfloat16)

        def zero_step(i, carry):
            for k in range(D // 32):
                rows_v[0, i, pl.ds(k * 32, 32)] = zeros32
            return carry

        with jax.named_scope("zero_acc"):
            lax.fori_loop(0, chunk, zero_step, 0)
            for k in range(RPT // chunk):
                pltpu.sync_copy(rows_v.at[0],
                                acc.at[pl.ds(s * RPT + k * chunk, chunk)])
            plsc.subcore_barrier()

        cnt = jnp.where(c == 0, cnt0, cnt1)
        base = c * (NS * cnt0) + s * cnt

        def run_phase(p):
            pltpu.sync_copy(src_hbm.at[pl.ds(base + p * slab, slab)], src_v)
            pltpu.sync_copy(dst_hbm.at[pl.ds(base + p * slab, slab)], dst_v)
            for b in range(nb):
                pltpu.async_copy(tbl_hbm.at[src_v.at[b]], rows_v.at[b],
                                 sems.at[b])

            def step(i, carry):
                j = i * nb
                for b in range(nb):
                    jj = j + b
                    pltpu.make_async_copy(tbl_hbm.at[src_v.at[jj]],
                                          rows_v.at[b], sems.at[b]).wait()
                    pltpu.sync_copy(rows_v.at[b], acc.at[dst_v.at[jj]],
                                    add=True)

                    @pl.when(jj + nb < slab)
                    def _():
                        pltpu.async_copy(tbl_hbm.at[src_v.at[jj + nb]],
                                         rows_v.at[b], sems.at[b])
                return carry

            lax.fori_loop(0, slab // nb, step, 0)

        with jax.named_scope("edges"):
            for p in range(nph):
                @pl.when(p * slab < cnt)
                def _():
                    run_phase(p)
            plsc.subcore_barrier()
        with jax.named_scope("dump"):
            for k in range(RPT // chunk):
                r = s * RPT + k * chunk
                pltpu.sync_copy(acc.at[pl.ds(r, chunk)],
                                out_hbm.at[c, pl.ds(r, chunk)])

    return pl.kernel(
        body,
        out_type=jax.ShapeDtypeStruct((NC, NP, D), jnp.bfloat16),
        mesh=plsc.VectorSubcoreMesh(core_axis_name="c", subcore_axis_name="s"),
        scratch_types=[
            pltpu.VMEM((slab, chunk), jnp.int32),
            pltpu.VMEM((slab, chunk), jnp.int32),
            pltpu.VMEM((nb, chunk, D), jnp.bfloat16),
            pltpu.VMEM_SHARED((NP, D), jnp.bfloat16),
            pltpu.SemaphoreType.DMA((nb,)),
        ],
        compiler_params=pltpu.CompilerParams(use_tc_tiling_on_sc=False),
    )


CH_AGG = 128
AGG_CNT0 = 144
AGG_CNT1 = 16
_agg_h = _make_agg(HID, CH_AGG, AGG_CNT0, AGG_CNT1, 16, 8)
_agg_c = _make_agg(CP, CH_AGG, AGG_CNT0, AGG_CNT1, 16, 8)


def _dsc_body(deg_ref, out_ref):
    deg = jnp.sum(deg_ref[...], axis=0, keepdims=True) + 1.0
    out_ref[...] = lax.rsqrt(deg)


def _l1_body(x_ref, w1_ref, out_ref):
    out_ref[...] = jnp.dot(x_ref[...], w1_ref[...],
                           preferred_element_type=jnp.float32)


def _scale_body(h_ref, dsc_ref, sh_ref):
    sh_ref[...] = (h_ref[...] * dsc_ref[...]).astype(jnp.bfloat16)


def _l2_body(agg_ref, sh_ref, dsc_ref, b1_ref, w2_ref, out_ref):
    dsc = dsc_ref[...]
    tot = (agg_ref[0].astype(jnp.float32) + agg_ref[1].astype(jnp.float32)
           + sh_ref[...].astype(jnp.float32))
    h1 = jnp.maximum(dsc * tot + b1_ref[...], 0.0)
    p = jnp.dot(h1 * dsc, w2_ref[...], preferred_element_type=jnp.float32)
    out_ref[...] = p.astype(jnp.bfloat16)


def _out_body(agg_ref, sp_ref, dsc_ref, b2_ref, out_ref):
    tot = (agg_ref[0].astype(jnp.float32) + agg_ref[1].astype(jnp.float32)
           + sp_ref[...].astype(jnp.float32))
    o = dsc_ref[...] * tot + b2_ref[...]
    logits = o[:, :C]
    m = jnp.max(logits, axis=1, keepdims=True)
    lse = jnp.log(jnp.sum(jnp.exp(logits - m), axis=1, keepdims=True))
    out_ref[...] = logits - m - lse


def _row_spec(d):
    return pl.BlockSpec((BR, d), lambda i: (i, 0))


def _full_spec(shape):
    return pl.BlockSpec(shape, lambda i: (0,) * len(shape))


def kernel(x, edge_index, W1, b1, W2, b2):
    src = edge_index[0].astype(jnp.int32)
    dst = edge_index[1].astype(jnp.int32)
    pad = EPAD - E
    srcp = jnp.concatenate([src, jnp.zeros((pad,), jnp.int32)])
    srcp = srcp.reshape(NW, NCH, CHUNK)
    dstp = jnp.concatenate([dst, jnp.full((pad,), N, jnp.int32)])
    dstp = dstp.reshape(NW, NCH, CHUNK)
    srcf = srcp.reshape(EPAD // CH_AGG, CH_AGG)
    dstf = dstp.reshape(EPAD // CH_AGG, CH_AGG)
    w2p = jnp.pad(W2, ((0, 0), (0, CP - C)))
    b1r = b1.reshape(1, HID)
    b2r = jnp.pad(b2, (0, CP - C)).reshape(1, CP)

    degp = _deg_call(dstp)
    h = pl.pallas_call(
        _l1_body,
        grid=(N // BR,),
        in_specs=[_row_spec(F), _full_spec((F, HID))],
        out_specs=_row_spec(HID),
        out_shape=jax.ShapeDtypeStruct((N, HID), jnp.float32),
    )(x, W1)
    dsc_row = pl.pallas_call(
        _dsc_body,
        out_shape=jax.ShapeDtypeStruct((1, NP), jnp.float32),
    )(degp)
    dsc = dsc_row.reshape(NP, 1)

    scaled_h = pl.pallas_call(
        _scale_body,
        grid=(N // BR,),
        in_specs=[_row_spec(HID), _row_spec(1)],
        out_specs=_row_spec(HID),
        out_shape=jax.ShapeDtypeStruct((N, HID), jnp.bfloat16),
    )(h, dsc)

    agg1 = _agg_h(scaled_h, srcf, dstf)

    scaled_p = pl.pallas_call(
        _l2_body,
        grid=(N // BR,),
        in_specs=[pl.BlockSpec((NC, BR, HID), lambda i: (0, i, 0)),
                  _row_spec(HID), _row_spec(1),
                  _full_spec((1, HID)), _full_spec((HID, CP))],
        out_specs=_row_spec(CP),
        out_shape=jax.ShapeDtypeStruct((N, CP), jnp.bfloat16),
    )(agg1, scaled_h, dsc, b1r, w2p)

    agg2 = _agg_c(scaled_p, srcf, dstf)

    out = pl.pallas_call(
        _out_body,
        grid=(N // BR,),
        in_specs=[pl.BlockSpec((NC, BR, CP), lambda i: (0, i, 0)),
                  _row_spec(CP), _row_spec(1),
                  _full_spec((1, CP))],
        out_specs=_row_spec(C),
        out_shape=jax.ShapeDtypeStruct((N, C), jnp.float32),
    )(agg2, scaled_p, dsc, b2r)
    return out


# speedup vs baseline: 1.2732x; 1.0004x over previous
import functools

import jax
import jax.numpy as jnp
from jax import lax
from jax.experimental import pallas as pl
from jax.experimental.pallas import tpu as pltpu
from jax.experimental.pallas import tpu_sc as plsc

N = 10000
E = 320000
F = 128
HID = 128
C = 40
CP = 64

NC = 2
NS = 16
NW = NC * NS
CHUNK = 64
NCH = 160
EPT = NCH * CHUNK
EPAD = EPT * NW
NP = 10240
RPT = NP // NS
NB = 8
SLAB = 32
TOTCH = EPT * NW // CHUNK
CNT0 = 288
CNT1 = 32
NPH0 = max(CNT0, CNT1) // SLAB

BR = 2000


def _deg_body(dst_hbm, out_hbm, idx_v, deg_v):
    c = lax.axis_index("c")
    s = lax.axis_index("s")
    w = c * NS + s
    pltpu.sync_copy(dst_hbm.at[w], idx_v)
    zeros16 = jnp.zeros((16,), jnp.float32)
    ones16 = jnp.ones((16,), jnp.float32)

    def zero_step(i, carry):
        deg_v[pl.ds(i * 16, 16)] = zeros16
        return carry

    lax.fori_loop(0, NP // 16, zero_step, 0)

    def count_step(j, carry):
        for k in range(CHUNK // 16):
            idx = idx_v[j, pl.ds(k * 16, 16)]
            plsc.addupdate_scatter(deg_v, [idx], ones16)
        return carry

    lax.fori_loop(0, NCH, count_step, 0)
    pltpu.sync_copy(deg_v, out_hbm.at[w])


_deg_call = functools.partial(
    pl.kernel,
    out_type=jax.ShapeDtypeStruct((NW, NP), jnp.float32),
    mesh=plsc.VectorSubcoreMesh(core_axis_name="c", subcore_axis_name="s"),
    scratch_types=[
        pltpu.VMEM((NCH, CHUNK), jnp.int32),
        pltpu.VMEM((NP,), jnp.float32),
    ],
    compiler_params=pltpu.CompilerParams(needs_layout_passes=False),
)(_deg_body)


def _make_agg(D, chunk, cnt0, cnt1, slab, nb, tc_tiling=False):
    nph = max(cnt0, cnt1) // slab

    def body(tbl_hbm, src_hbm, dst_hbm, out_hbm, src_v, dst_v, rows_v, acc,
             sems):
        c = lax.axis_index("c")
        s = lax.axis_index("s")

        zeros2x16 = jnp.zeros((2, 16), jnp.bfloat16)

        def zero_step(i, carry):
            for k in range(D // 16):
                rows_v[0, pl.ds(i * 2, 2), pl.ds(k * 16, 16)] = zeros2x16
            return carry

        with jax.named_scope("zero_acc"):
            lax.fori_loop(0, chunk // 2, zero_step, 0)
            for k in range(RPT // chunk):
                pltpu.sync_copy(rows_v.at[0],
                                acc.at[pl.ds(s * RPT + k * chunk, chunk)])
            plsc.subcore_barrier()

        cnt = jnp.where(c == 0, cnt0, cnt1)
        base = c * (NS * cnt0) + s * cnt

        def run_phase(p):
            pltpu.sync_copy(src_hbm.at[pl.ds(base + p * slab, slab)], src_v)
            pltpu.sync_copy(dst_hbm.at[pl.ds(base + p * slab, slab)], dst_v)
            for b in range(nb):
                pltpu.async_copy(tbl_hbm.at[src_v.at[b]], rows_v.at[b],
                                 sems.at[b])

            def step(i, carry):
                j = i * nb
                for b in range(nb):
                    jj = j + b
                    pltpu.make_async_copy(tbl_hbm.at[src_v.at[jj]],
                                          rows_v.at[b], sems.at[b]).wait()
                    pltpu.sync_copy(rows_v.at[b], acc.at[dst_v.at[jj]],
                                    add=True)

                    @pl.when(jj + nb < slab)
                    def _():
                        pltpu.async_copy(tbl_hbm.at[src_v.at[jj + nb]],
                                         rows_v.at[b], sems.at[b])
                return carry

            lax.fori_loop(0, slab // nb, step, 0)

        with jax.named_scope("edges"):
            for p in range(nph):
                @pl.when(p * slab < cnt)
                def _():
                    run_phase(p)
            plsc.subcore_barrier()
        with jax.named_scope("dump"):
            for k in range(RPT // chunk):
                r = s * RPT + k * chunk
                pltpu.sync_copy(acc.at[pl.ds(r, chunk)],
                                out_hbm.at[c, pl.ds(r, chunk)])

    return pl.kernel(
        body,
        out_type=jax.ShapeDtypeStruct((NC, NP, D), jnp.bfloat16),
        mesh=plsc.VectorSubcoreMesh(core_axis_name="c", subcore_axis_name="s"),
        scratch_types=[
            pltpu.VMEM((slab, chunk), jnp.int32),
            pltpu.VMEM((slab, chunk), jnp.int32),
            pltpu.VMEM((nb, chunk, D), jnp.bfloat16),
            pltpu.VMEM_SHARED((NP, D), jnp.bfloat16),
            pltpu.SemaphoreType.DMA((nb,)),
        ],
        compiler_params=pltpu.CompilerParams(use_tc_tiling_on_sc=tc_tiling),
    )


CH_AGG = 128
AGG_CNT0 = 144
AGG_CNT1 = 16
_agg_h = _make_agg(HID, CH_AGG, AGG_CNT0, AGG_CNT1, 16, 8)
_agg_c = _make_agg(CP, CH_AGG, AGG_CNT0, AGG_CNT1, 16, 8)


def _dsc_body(deg_ref, out_ref):
    deg = jnp.sum(deg_ref[...], axis=0, keepdims=True) + 1.0
    out_ref[...] = lax.rsqrt(deg)


def _l1_body(x_ref, w1_ref, out_ref):
    out_ref[...] = jnp.dot(x_ref[...], w1_ref[...],
                           preferred_element_type=jnp.float32)


def _scale_body(h_ref, dsc_ref, sh_ref):
    sh_ref[...] = (h_ref[...] * dsc_ref[...]).astype(jnp.bfloat16)


def _l2_body(agg_ref, sh_ref, dsc_ref, b1_ref, w2_ref, out_ref):
    dsc = dsc_ref[...]
    tot = (agg_ref[0].astype(jnp.float32) + agg_ref[1].astype(jnp.float32)
           + sh_ref[...].astype(jnp.float32))
    h1 = jnp.maximum(dsc * tot + b1_ref[...], 0.0)
    p = jnp.dot(h1 * dsc, w2_ref[...], preferred_element_type=jnp.float32)
    out_ref[...] = p.astype(jnp.bfloat16)


def _out_body(agg_ref, sp_ref, dsc_ref, b2_ref, out_ref):
    tot = (agg_ref[0].astype(jnp.float32) + agg_ref[1].astype(jnp.float32)
           + sp_ref[...].astype(jnp.float32))
    o = dsc_ref[...] * tot + b2_ref[...]
    logits = o[:, :C]
    m = jnp.max(logits, axis=1, keepdims=True)
    lse = jnp.log(jnp.sum(jnp.exp(logits - m), axis=1, keepdims=True))
    out_ref[...] = logits - m - lse


def _row_spec(d):
    return pl.BlockSpec((BR, d), lambda i: (i, 0))


def _full_spec(shape):
    return pl.BlockSpec(shape, lambda i: (0,) * len(shape))


def kernel(x, edge_index, W1, b1, W2, b2):
    src = edge_index[0].astype(jnp.int32)
    dst = edge_index[1].astype(jnp.int32)
    pad = EPAD - E
    srcp = jnp.concatenate([src, jnp.zeros((pad,), jnp.int32)])
    srcp = srcp.reshape(NW, NCH, CHUNK)
    dstp = jnp.concatenate([dst, jnp.full((pad,), N, jnp.int32)])
    dstp = dstp.reshape(NW, NCH, CHUNK)
    srcf = srcp.reshape(EPAD // CH_AGG, CH_AGG)
    dstf = dstp.reshape(EPAD // CH_AGG, CH_AGG)
    w2p = jnp.pad(W2, ((0, 0), (0, CP - C)))
    b1r = b1.reshape(1, HID)
    b2r = jnp.pad(b2, (0, CP - C)).reshape(1, CP)

    degp = _deg_call(dstp)
    h = pl.pallas_call(
        _l1_body,
        grid=(N // BR,),
        in_specs=[_row_spec(F), _full_spec((F, HID))],
        out_specs=_row_spec(HID),
        out_shape=jax.ShapeDtypeStruct((N, HID), jnp.float32),
    )(x, W1)
    dsc_row = pl.pallas_call(
        _dsc_body,
        out_shape=jax.ShapeDtypeStruct((1, NP), jnp.float32),
    )(degp)
    dsc = dsc_row.reshape(NP, 1)

    scaled_h = pl.pallas_call(
        _scale_body,
        grid=(N // BR,),
        in_specs=[_row_spec(HID), _row_spec(1)],
        out_specs=_row_spec(HID),
        out_shape=jax.ShapeDtypeStruct((N, HID), jnp.bfloat16),
    )(h, dsc)

    agg1 = _agg_h(scaled_h, srcf, dstf)

    scaled_p = pl.pallas_call(
        _l2_body,
        grid=(N // BR,),
        in_specs=[pl.BlockSpec((NC, BR, HID), lambda i: (0, i, 0)),
                  _row_spec(HID), _row_spec(1),
                  _full_spec((1, HID)), _full_spec((HID, CP))],
        out_specs=_row_spec(CP),
        out_shape=jax.ShapeDtypeStruct((N, CP), jnp.bfloat16),
    )(agg1, scaled_h, dsc, b1r, w2p)

    agg2 = _agg_c(scaled_p, srcf, dstf)

    out = pl.pallas_call(
        _out_body,
        grid=(N // BR,),
        in_specs=[pl.BlockSpec((NC, BR, CP), lambda i: (0, i, 0)),
                  _row_spec(CP), _row_spec(1),
                  _full_spec((1, CP))],
        out_specs=_row_spec(C),
        out_shape=jax.ShapeDtypeStruct((N, C), jnp.float32),
    )(agg2, scaled_p, dsc, b2r)
    return out
